# baseline (speedup 1.0000x reference)
import functools

import jax
import jax.numpy as jnp
from jax import lax
from jax.experimental import pallas as pl
from jax.experimental.pallas import tpu as pltpu
from jax.experimental.pallas import tpu_sc as plsc

_N = 32768
_D = 768
_E = 64
_BN = 512

_NW = 32
_ROWS_PER_W = _N // _NW
_BLK = 128
_NBLK = _ROWS_PER_W // _BLK
_G = 4
_LANES = 16


def _tc_probs_body(x_ref, g_ref, b_ref, w1_ref, b1_ref, w2_ref, b2_ref,
                   probs_ref):
    x = x_ref[...]
    mu = jnp.mean(x, axis=-1, keepdims=True)
    xc = x - mu
    var = jnp.mean(xc * xc, axis=-1, keepdims=True)
    xn = xc * jax.lax.rsqrt(var + 1e-5) * g_ref[...] + b_ref[...]

    h = jnp.dot(xn, w1_ref[...], preferred_element_type=jnp.float32)
    h = h + b1_ref[...]
    h = 0.5 * h * (1.0 + jax.lax.erf(h * 0.7071067811865476))

    logits = jnp.dot(h, w2_ref[...], preferred_element_type=jnp.float32)
    logits = logits + b2_ref[...]

    m = jnp.max(logits, axis=-1, keepdims=True)
    ex = jnp.exp(logits - m)
    probs_ref[...] = ex / jnp.sum(ex, axis=-1, keepdims=True)


def _tc_probs(fused_latent, ln_g, ln_b, W1, b1, W2, b2):
    grid = (_N // _BN,)
    return pl.pallas_call(
        _tc_probs_body,
        grid=grid,
        in_specs=[
            pl.BlockSpec((_BN, _D), lambda i: (i, 0)),
            pl.BlockSpec((1, _D), lambda i: (0, 0)),
            pl.BlockSpec((1, _D), lambda i: (0, 0)),
            pl.BlockSpec((_D, _D), lambda i: (0, 0)),
            pl.BlockSpec((1, _D), lambda i: (0, 0)),
            pl.BlockSpec((_D, _E), lambda i: (0, 0)),
            pl.BlockSpec((1, _E), lambda i: (0, 0)),
        ],
        out_specs=pl.BlockSpec((_BN, _E), lambda i: (i, 0)),
        out_shape=jax.ShapeDtypeStruct((_N, _E), jnp.float32),
        compiler_params=pltpu.CompilerParams(
            dimension_semantics=("arbitrary",),
        ),
    )(fused_latent, ln_g.reshape(1, _D), ln_b.reshape(1, _D),
      W1, b1.reshape(1, _D), W2, b2.reshape(1, _E))


def _scan_top2(in_tile, rows):
    neg = jnp.full((_LANES,), -1.0, dtype=jnp.float32)
    zero_i = jnp.zeros((_LANES,), dtype=jnp.int32)

    def body(e, carry):
        p1, p2, i1, i2 = carry
        e_vec = jnp.full((_LANES,), e, dtype=jnp.int32)
        v = plsc.load_gather(in_tile, [rows, e_vec])
        gt1 = v > p1
        gt2 = v > p2
        p2n = jnp.where(gt1, p1, jnp.where(gt2, v, p2))
        i2n = jnp.where(gt1, i1, jnp.where(gt2, e_vec, i2))
        p1n = jnp.where(gt1, v, p1)
        i1n = jnp.where(gt1, e_vec, i1)
        return p1n, p2n, i1n, i2n

    return lax.fori_loop(0, _E, body, (neg, neg, zero_i, zero_i))


def _sc_route_kernel(probs_hbm, zeros_hbm, routed_hbm, idx_hbm,
                     in_t, out_t, idx_t):
    info = plsc.get_sparse_core_info()
    wid = lax.axis_index("s") * info.num_cores + lax.axis_index("c")
    pltpu.sync_copy(zeros_hbm, out_t)

    col0 = jnp.zeros((_LANES,), dtype=jnp.int32)
    col1 = jnp.ones((_LANES,), dtype=jnp.int32)
    zf = jnp.zeros((_LANES,), dtype=jnp.float32)
    eps = jnp.full((_LANES,), 1e-8, dtype=jnp.float32)

    for blk in range(_NBLK):
        base = wid * _ROWS_PER_W + blk * _BLK
        pltpu.sync_copy(probs_hbm.at[pl.ds(base, _BLK)], in_t)
        touched = []
        for grp in range(_BLK // _LANES):
            rows = lax.iota(jnp.int32, _LANES) + (grp * _LANES)
            p1, p2, i1, i2 = _scan_top2(in_t, rows)
            denom = p1 + p2 + eps
            rden = 1.0 / denom
            plsc.store_scatter(out_t, [rows, i1], p1 * rden)
            plsc.store_scatter(out_t, [rows, i2], p2 * rden)
            plsc.store_scatter(idx_t, [rows, col0], i1)
            plsc.store_scatter(idx_t, [rows, col1], i2)
            touched.append((rows, i1, i2))
        pltpu.sync_copy(out_t, routed_hbm.at[pl.ds(base, _BLK)])
        pltpu.sync_copy(idx_t, idx_hbm.at[pl.ds(base, _BLK)])
        for rows, i1, i2 in touched:
            plsc.store_scatter(out_t, [rows, i1], zf)
            plsc.store_scatter(out_t, [rows, i2], zf)


@functools.partial(
    pl.kernel,
    mesh=plsc.VectorSubcoreMesh(core_axis_name="c", subcore_axis_name="s"),
    out_type=[
        jax.ShapeDtypeStruct((_N, _E), jnp.float32),
        jax.ShapeDtypeStruct((_N, 2), jnp.int32),
    ],
    scratch_types=[
        pltpu.VMEM((_BLK, _E), jnp.float32),
        pltpu.VMEM((_BLK, _E), jnp.float32),
        pltpu.VMEM((_BLK, 2), jnp.int32),
    ],
    compiler_params=pltpu.CompilerParams(needs_layout_passes=False),
)
def _sc_route(probs_hbm, zeros_hbm, routed_hbm, idx_hbm, in_t, out_t, idx_t):
    _sc_route_kernel(probs_hbm, zeros_hbm, routed_hbm, idx_hbm,
                     in_t, out_t, idx_t)


def kernel(fused_latent, ln_g, ln_b, W1, b1, W2, b2):
    probs = _tc_probs(fused_latent, ln_g, ln_b, W1, b1, W2, b2)
    zeros = jnp.zeros((_BLK, _E), dtype=jnp.float32)
    routed, idx = _sc_route(probs, zeros)
    return routed, idx


# speedup vs baseline: 1.0538x; 1.0538x over previous
import functools

import jax
import jax.numpy as jnp
from jax import lax
from jax.experimental import pallas as pl
from jax.experimental.pallas import tpu as pltpu
from jax.experimental.pallas import tpu_sc as plsc

_N = 32768
_D = 768
_E = 64
_BN = 512

_NW = 32
_ROWS_PER_W = _N // _NW
_BLK = 128
_NBLK = _ROWS_PER_W // _BLK
_G = 4
_LANES = 16


def _tc_probs_body(x_ref, g_ref, b_ref, w1_ref, b1_ref, w2_ref, b2_ref,
                   probs_ref):
    x = x_ref[...]
    mu = jnp.mean(x, axis=-1, keepdims=True)
    xc = x - mu
    var = jnp.mean(xc * xc, axis=-1, keepdims=True)
    xn = xc / jnp.sqrt(var + 1e-5) * g_ref[...] + b_ref[...]

    h = jnp.dot(xn, w1_ref[...], preferred_element_type=jnp.float32)
    h = h + b1_ref[...]
    h = 0.5 * h * (1.0 + jax.lax.erf(h * 0.7071067811865476))

    logits = jnp.dot(h, w2_ref[...], preferred_element_type=jnp.float32)
    logits = logits + b2_ref[...]

    m = jnp.max(logits, axis=-1, keepdims=True)
    ex = jnp.exp(logits - m)
    probs_ref[...] = ex / jnp.sum(ex, axis=-1, keepdims=True)


def _tc_probs(fused_latent, ln_g, ln_b, W1, b1, W2, b2):
    grid = (_N // _BN,)
    return pl.pallas_call(
        _tc_probs_body,
        grid=grid,
        in_specs=[
            pl.BlockSpec((_BN, _D), lambda i: (i, 0)),
            pl.BlockSpec((1, _D), lambda i: (0, 0)),
            pl.BlockSpec((1, _D), lambda i: (0, 0)),
            pl.BlockSpec((_D, _D), lambda i: (0, 0)),
            pl.BlockSpec((1, _D), lambda i: (0, 0)),
            pl.BlockSpec((_D, _E), lambda i: (0, 0)),
            pl.BlockSpec((1, _E), lambda i: (0, 0)),
        ],
        out_specs=pl.BlockSpec((_BN, _E), lambda i: (i, 0)),
        out_shape=jax.ShapeDtypeStruct((_N, _E), jnp.float32),
        compiler_params=pltpu.CompilerParams(
            dimension_semantics=("arbitrary",),
        ),
    )(fused_latent, ln_g.reshape(1, _D), ln_b.reshape(1, _D),
      W1, b1.reshape(1, _D), W2, b2.reshape(1, _E))


def _scan_top2_groups(in_tile, rows_list):
    ng = len(rows_list)
    neg = jnp.full((_LANES,), -1.0, dtype=jnp.float32)
    zero_i = jnp.zeros((_LANES,), dtype=jnp.int32)

    def body(e, carry):
        e_vec = jnp.full((_LANES,), e, dtype=jnp.int32)
        out = []
        for g in range(ng):
            p1, p2, i1, i2 = carry[g]
            v = plsc.load_gather(in_tile, [rows_list[g], e_vec])
            gt1 = v > p1
            gt2 = v > p2
            p2n = jnp.where(gt1, p1, jnp.where(gt2, v, p2))
            i2n = jnp.where(gt1, i1, jnp.where(gt2, e_vec, i2))
            p1n = jnp.where(gt1, v, p1)
            i1n = jnp.where(gt1, e_vec, i1)
            out.append((p1n, p2n, i1n, i2n))
        return tuple(out)

    init = tuple((neg, neg, zero_i, zero_i) for _ in range(ng))
    return lax.fori_loop(0, _E, body, init)


def _sc_route_kernel(probs_hbm, zeros_hbm, routed_hbm, idx_hbm,
                     in_t, out_t, idx_t):
    info = plsc.get_sparse_core_info()
    wid = lax.axis_index("s") * info.num_cores + lax.axis_index("c")
    pltpu.sync_copy(zeros_hbm, out_t)

    col0 = jnp.zeros((_LANES,), dtype=jnp.int32)
    col1 = jnp.ones((_LANES,), dtype=jnp.int32)
    zf = jnp.zeros((_LANES,), dtype=jnp.float32)
    eps = jnp.full((_LANES,), 1e-8, dtype=jnp.float32)

    for blk in range(_NBLK):
        base = wid * _ROWS_PER_W + blk * _BLK
        pltpu.sync_copy(probs_hbm.at[pl.ds(base, _BLK)], in_t)
        rows_list = [lax.iota(jnp.int32, _LANES) + (grp * _LANES)
                     for grp in range(_BLK // _LANES)]
        results = _scan_top2_groups(in_t, rows_list)
        touched = []
        for rows, (p1, p2, i1, i2) in zip(rows_list, results):
            denom = p1 + p2 + eps
            plsc.store_scatter(out_t, [rows, i1], p1 / denom)
            plsc.store_scatter(out_t, [rows, i2], p2 / denom)
            plsc.store_scatter(idx_t, [rows, col0], i1)
            plsc.store_scatter(idx_t, [rows, col1], i2)
            touched.append((rows, i1, i2))
        pltpu.sync_copy(out_t, routed_hbm.at[pl.ds(base, _BLK)])
        pltpu.sync_copy(idx_t, idx_hbm.at[pl.ds(base, _BLK)])
        for rows, i1, i2 in touched:
            plsc.store_scatter(out_t, [rows, i1], zf)
            plsc.store_scatter(out_t, [rows, i2], zf)


@functools.partial(
    pl.kernel,
    mesh=plsc.VectorSubcoreMesh(core_axis_name="c", subcore_axis_name="s"),
    out_type=[
        jax.ShapeDtypeStruct((_N, _E), jnp.float32),
        jax.ShapeDtypeStruct((_N, 2), jnp.int32),
    ],
    scratch_types=[
        pltpu.VMEM((_BLK, _E), jnp.float32),
        pltpu.VMEM((_BLK, _E), jnp.float32),
        pltpu.VMEM((_BLK, 2), jnp.int32),
    ],
    compiler_params=pltpu.CompilerParams(needs_layout_passes=False),
)
def _sc_route(probs_hbm, zeros_hbm, routed_hbm, idx_hbm, in_t, out_t, idx_t):
    _sc_route_kernel(probs_hbm, zeros_hbm, routed_hbm, idx_hbm,
                     in_t, out_t, idx_t)


def kernel(fused_latent, ln_g, ln_b, W1, b1, W2, b2):
    probs = _tc_probs(fused_latent, ln_g, ln_b, W1, b1, W2, b2)
    zeros = jnp.zeros((_BLK, _E), dtype=jnp.float32)
    routed, idx = _sc_route(probs, zeros)
    return routed, idx
